# baseline (speedup 1.0000x reference)
import functools

import jax
import jax.numpy as jnp
from jax import lax
from jax.experimental import pallas as pl
from jax.experimental.pallas import tpu as pltpu
from jax.experimental.pallas import tpu_sc as plsc

_B = 16384
_D = 64
_LANES = 128
_W = 256
_SH = 8
_MAXOFF = 7813 * _LANES - _W
_NC = 2
_NS = 16
_NW = _NC * _NS
_BPW = _B // _NW
_RING = 3


@functools.cache
def _build():
    mesh = plsc.VectorSubcoreMesh(core_axis_name="c", subcore_axis_name="s")

    @functools.partial(
        pl.kernel,
        mesh=mesh,
        out_type=jax.ShapeDtypeStruct((_B, _D), jnp.float32),
        scratch_types=[
            pltpu.VMEM((_BPW + 16,), jnp.int32),
            pltpu.VMEM((_BPW + 16,), jnp.int32),
            pltpu.VMEM((_RING, _D, _W), jnp.float32),
            pltpu.VMEM((_BPW, _D), jnp.float32),
        ] + [pltpu.SemaphoreType.DMA] * (_RING + 1) + [
        ],
        compiler_params=pltpu.CompilerParams(needs_layout_passes=False),
    )
    def gather_kernel(tableT_hbm, sidx_hbm, opos_hbm, out_hbm,
                      sidx_v, opos_v, ring_v, rows_v, *sems_all):
        sems = list(sems_all[:_RING])
        semw = sems_all[_RING]
        wid = lax.axis_index("s") * _NC + lax.axis_index("c")
        base = wid * _BPW
        pltpu.sync_copy(sidx_hbm.at[pl.ds(base, _BPW)],
                        sidx_v.at[pl.ds(0, _BPW)])
        pltpu.sync_copy(opos_hbm.at[pl.ds(base, _BPW)],
                        opos_v.at[pl.ds(0, _BPW)])

        blo = sidx_v[pl.ds(0, 16)][0] >> _SH
        bhi = sidx_v[pl.ds(_BPW - 16, 16)][15] >> _SH

        lane16 = jnp.arange(16, dtype=jnp.int32)

        def start(b, i):
            off = pl.multiple_of(
                jnp.minimum(b * _W, _MAXOFF), _LANES)
            pltpu.async_copy(tableT_hbm.at[:, pl.ds(off, _W)],
                             ring_v.at[i], sems[i])

        def wait(i):
            pltpu.make_async_copy(
                tableT_hbm.at[:, pl.ds(0, _W)], ring_v.at[i],
                sems[i]).wait()

        def proc(i, b, q0):
            blk = ring_v.at[i]
            off = jnp.minimum(b * _W, _MAXOFF)
            lim = (b + 1) * _W

            def cond(q):
                v = sidx_v[pl.ds(q, 16)][0]
                return jnp.logical_and(q < _BPW, v < lim)

            def body(q):
                v = sidx_v[pl.ds(q, 16)][0]
                o = opos_v[pl.ds(q, 16)][0]
                lvec = jnp.full((16,), 0, jnp.int32) + (v - off)
                for g in range(4):
                    col = plsc.load_gather(
                        blk, [lane16 + g * 16, lvec])
                    rows_v[q, pl.ds(g * 16, 16)] = col
                pltpu.async_copy(rows_v.at[q], out_hbm.at[o], semw)
                return q + 1

            return lax.while_loop(cond, body, q0)

        for i in range(_RING):
            @pl.when(blo + i <= bhi)
            def _(i=i):
                start(blo + i, i)

        def rotate(k, q):
            b0 = blo + k * _RING
            for i in range(_RING):
                b = b0 + i

                @pl.when(b <= bhi)
                def _(i=i):
                    wait(i)

                q = proc(i, b, q)

                @pl.when(b + _RING <= bhi)
                def _(i=i, b=b):
                    start(b + _RING, i)

            return q

        nrot = (bhi - blo + _RING) // _RING
        lax.fori_loop(0, nrot, rotate, 0)

        pltpu.make_async_copy(
            out_hbm.at[pl.ds(0, _BPW)], rows_v, semw).wait()

    return gather_kernel


def kernel(memory, index, t, current_event_id):
    idx = index.astype(jnp.int32)
    pos = jnp.arange(_B, dtype=jnp.int32)
    sidx, opos = lax.sort((idx, pos), num_keys=1)
    return _build()(memory.T, sidx, opos)


# speedup vs baseline: 1.1173x; 1.1173x over previous
import functools

import jax
import jax.numpy as jnp
from jax import lax
from jax.experimental import pallas as pl
from jax.experimental.pallas import tpu as pltpu
from jax.experimental.pallas import tpu_sc as plsc

_B = 16384
_D = 64
_LANES = 128
_W = 128
_SH = 7
_MAXOFF = 7813 * _LANES - _W
_NC = 2
_NS = 16
_NW = _NC * _NS
_BPW = _B // _NW
_RING = 7


@functools.cache
def _build():
    mesh = plsc.VectorSubcoreMesh(core_axis_name="c", subcore_axis_name="s")

    @functools.partial(
        pl.kernel,
        mesh=mesh,
        out_type=jax.ShapeDtypeStruct((_B, _D), jnp.float32),
        scratch_types=[
            pltpu.VMEM((_BPW + 16,), jnp.int32),
            pltpu.VMEM((_BPW + 16,), jnp.int32),
            pltpu.VMEM((_RING, _D, _W), jnp.float32),
            pltpu.VMEM((_BPW, _D), jnp.float32),
        ] + [pltpu.SemaphoreType.DMA] * (_RING + 1) + [
        ],
        compiler_params=pltpu.CompilerParams(needs_layout_passes=False),
    )
    def gather_kernel(tableT_hbm, sidx_hbm, opos_hbm, out_hbm,
                      sidx_v, opos_v, ring_v, rows_v, *sems_all):
        sems = list(sems_all[:_RING])
        semw = sems_all[_RING]
        wid = lax.axis_index("s") * _NC + lax.axis_index("c")
        base = wid * _BPW
        pltpu.sync_copy(sidx_hbm.at[pl.ds(base, _BPW)],
                        sidx_v.at[pl.ds(0, _BPW)])
        pltpu.sync_copy(opos_hbm.at[pl.ds(base, _BPW)],
                        opos_v.at[pl.ds(0, _BPW)])

        blo = sidx_v[pl.ds(0, 16)][0] >> _SH
        bhi = sidx_v[pl.ds(_BPW - 16, 16)][15] >> _SH

        lane16 = jnp.arange(16, dtype=jnp.int32)

        def start(b, i):
            off = pl.multiple_of(
                jnp.minimum(b * _W, _MAXOFF), _LANES)
            pltpu.async_copy(tableT_hbm.at[:, pl.ds(off, _W)],
                             ring_v.at[i], sems[i])

        def wait(i):
            pltpu.make_async_copy(
                tableT_hbm.at[:, pl.ds(0, _W)], ring_v.at[i],
                sems[i]).wait()

        def proc(i, b, q0):
            blk = ring_v.at[i]
            off = jnp.minimum(b * _W, _MAXOFF)
            lim = (b + 1) * _W

            def cond(q):
                v = sidx_v[pl.ds(q, 16)][0]
                return jnp.logical_and(q < _BPW, v < lim)

            def body(q):
                v = sidx_v[pl.ds(q, 16)][0]
                o = opos_v[pl.ds(q, 16)][0]
                lvec = jnp.full((16,), 0, jnp.int32) + (v - off)
                for g in range(4):
                    col = plsc.load_gather(
                        blk, [lane16 + g * 16, lvec])
                    rows_v[q, pl.ds(g * 16, 16)] = col
                pltpu.async_copy(rows_v.at[q], out_hbm.at[o], semw)
                return q + 1

            return lax.while_loop(cond, body, q0)

        for i in range(_RING):
            @pl.when(blo + i <= bhi)
            def _(i=i):
                start(blo + i, i)

        def rotate(k, q):
            b0 = blo + k * _RING
            for i in range(_RING):
                b = b0 + i

                @pl.when(b <= bhi)
                def _(i=i):
                    wait(i)

                q = proc(i, b, q)

                @pl.when(b + _RING <= bhi)
                def _(i=i, b=b):
                    start(b + _RING, i)

            return q

        nrot = (bhi - blo + _RING) // _RING
        lax.fori_loop(0, nrot, rotate, 0)

        pltpu.make_async_copy(
            out_hbm.at[pl.ds(0, _BPW)], rows_v, semw).wait()

    return gather_kernel


def kernel(memory, index, t, current_event_id):
    idx = index.astype(jnp.int32)
    pos = jnp.arange(_B, dtype=jnp.int32)
    sidx, opos = lax.sort((idx, pos), num_keys=1)
    return _build()(memory.T, sidx, opos)


# speedup vs baseline: 1.2376x; 1.1077x over previous
import functools

import jax
import jax.numpy as jnp
from jax import lax
from jax.experimental import pallas as pl
from jax.experimental.pallas import tpu as pltpu
from jax.experimental.pallas import tpu_sc as plsc

_B = 16384
_D = 64
_LANES = 128
_W = 128
_SH = 7
_MAXOFF = 7813 * _LANES - _W
_NC = 2
_NS = 16
_NW = _NC * _NS
_BPW = _B // _NW
_RING = 7


@functools.cache
def _build():
    mesh = plsc.VectorSubcoreMesh(core_axis_name="c", subcore_axis_name="s")

    @functools.partial(
        pl.kernel,
        mesh=mesh,
        out_type=jax.ShapeDtypeStruct((_B, _D), jnp.float32),
        scratch_types=[
            pltpu.VMEM((_BPW + 32,), jnp.int32),
            pltpu.VMEM((_BPW + 16,), jnp.int32),
            pltpu.VMEM((_BPW + 32,), jnp.int32),
            pltpu.VMEM((_RING, _D, _W), jnp.float32),
            pltpu.VMEM((_BPW, _D), jnp.float32),
        ] + [pltpu.SemaphoreType.DMA] * (_RING + 1) + [
        ],
        compiler_params=pltpu.CompilerParams(needs_layout_passes=False),
    )
    def gather_kernel(tableT_hbm, sidx_hbm, opos_hbm, out_hbm,
                      sidx_v, opos_v, blist_v, ring_v, rows_v, *sems_all):
        sems = list(sems_all[:_RING])
        semw = sems_all[_RING]
        wid = lax.axis_index("s") * _NC + lax.axis_index("c")
        base = wid * _BPW
        sidx_v[pl.ds(0, 16)] = jnp.full((16,), -1, jnp.int32)
        pltpu.sync_copy(sidx_hbm.at[pl.ds(base, _BPW)],
                        sidx_v.at[pl.ds(16, _BPW)])
        pltpu.sync_copy(opos_hbm.at[pl.ds(base, _BPW)],
                        opos_v.at[pl.ds(0, _BPW)])

        lane16 = jnp.arange(16, dtype=jnp.int32)

        nblk = jnp.int32(0)
        for k in range(_BPW // 16):
            bvec = sidx_v[pl.ds(16 + k * 16, 16)] >> _SH
            shifted = sidx_v[pl.ds(15 + k * 16, 16)] >> _SH
            keep = bvec != shifted
            plsc.store_compressed(blist_v.at[pl.ds(nblk, 16)], bvec, mask=keep)
            nblk = nblk + jnp.sum(keep.astype(jnp.int32))

        def start(b, i):
            off = pl.multiple_of(
                jnp.minimum(b * _W, _MAXOFF), _LANES)
            pltpu.async_copy(tableT_hbm.at[:, pl.ds(off, _W)],
                             ring_v.at[i], sems[i])

        def wait(i):
            pltpu.make_async_copy(
                tableT_hbm.at[:, pl.ds(0, _W)], ring_v.at[i],
                sems[i]).wait()

        def proc(i, b, q0):
            blk = ring_v.at[i]
            off = jnp.minimum(b * _W, _MAXOFF)
            lim = (b + 1) * _W

            def cond(q):
                v = sidx_v[pl.ds(q + 16, 16)][0]
                return jnp.logical_and(q < _BPW, v < lim)

            def body(q):
                v = sidx_v[pl.ds(q + 16, 16)][0]
                o = opos_v[pl.ds(q, 16)][0]
                lvec = jnp.full((16,), 0, jnp.int32) + (v - off)
                for g in range(4):
                    col = plsc.load_gather(
                        blk, [lane16 + g * 16, lvec])
                    rows_v[q, pl.ds(g * 16, 16)] = col
                pltpu.async_copy(rows_v.at[q], out_hbm.at[o], semw)
                return q + 1

            return lax.while_loop(cond, body, q0)

        def blk_at(j):
            return blist_v[pl.ds(j, 16)][0]

        for i in range(_RING):
            @pl.when(i < nblk)
            def _(i=i):
                start(blk_at(jnp.int32(i)), i)

        def rotate(k, q):
            j0 = k * _RING
            for i in range(_RING):
                j = j0 + i

                @pl.when(j < nblk)
                def _(i=i, j=j):
                    wait(i)

                q = lax.cond(
                    j < nblk,
                    lambda qq, i=i, j=j: proc(i, blk_at(j), qq),
                    lambda qq: qq,
                    q)

                @pl.when(j + _RING < nblk)
                def _(i=i, j=j):
                    start(blk_at(j + _RING), i)

            return q

        nrot = (nblk + _RING - 1) // _RING
        lax.fori_loop(0, nrot, rotate, 0)

        pltpu.make_async_copy(
            out_hbm.at[pl.ds(0, _BPW)], rows_v, semw).wait()

    return gather_kernel


def kernel(memory, index, t, current_event_id):
    idx = index.astype(jnp.int32)
    pos = jnp.arange(_B, dtype=jnp.int32)
    sidx, opos = lax.sort((idx, pos), num_keys=1)
    return _build()(memory.T, sidx, opos)
